# Initial kernel scaffold; baseline (speedup 1.0000x reference)
import jax
import jax.numpy as jnp
from jax.experimental import pallas as pl


def kernel(inputs, targets, feature_memory, k, features):
    raise NotImplementedError("write your pallas kernel here")


# speedup vs baseline: 1.8535x; 1.8535x over previous
import functools

import jax
import jax.numpy as jnp
from jax.experimental import pallas as pl
from jax.experimental.pallas import tpu as pltpu

NUM_SAMPLES = 100000
NUM_FEATURES = 64
BATCH = 1024
TEMP = 0.05
BLOCK_N = 2000
NUM_BLOCKS = NUM_SAMPLES // BLOCK_N


def _partial_kernel(x_ref, t_ref, f_ref, s_out_ref, t_out_ref):
    j = pl.program_id(0)
    x = x_ref[...]
    ss = jnp.sum(x * x, axis=1, keepdims=True)
    xn = x / jnp.maximum(jnp.sqrt(ss), 1e-12)
    xb = xn.astype(jnp.bfloat16)
    fb = f_ref[...]
    logits = jax.lax.dot_general(
        xb, fb, (((1,), (1,)), ((), ())),
        preferred_element_type=jnp.float32) * (1.0 / TEMP)
    e = jnp.exp(logits)
    s_out_ref[0, 0, :] = jnp.sum(e, axis=1)
    col = jax.lax.broadcasted_iota(jnp.int32, (BATCH, BLOCK_N), 1) + j * BLOCK_N
    mask = col == t_ref[...]
    t_out_ref[0, 0, :] = jnp.sum(jnp.where(mask, logits, 0.0), axis=1)


def _reduce_kernel(s_ref, t_ref, out_ref):
    sumexp = jnp.sum(s_ref[...], axis=0)
    tsum = jnp.sum(t_ref[...], axis=0)
    loss = jnp.mean(jnp.log(sumexp) - tsum)
    out_ref[...] = loss.reshape(1, 1)


@jax.jit
def kernel(inputs, targets, feature_memory, k, features):
    del feature_memory, k
    fb16 = features.astype(jnp.bfloat16)
    t2d = targets.astype(jnp.int32).reshape(BATCH, 1)
    s_part, t_part = pl.pallas_call(
        _partial_kernel,
        grid=(NUM_BLOCKS,),
        in_specs=[
            pl.BlockSpec((BATCH, NUM_FEATURES), lambda j: (0, 0)),
            pl.BlockSpec((BATCH, 1), lambda j: (0, 0)),
            pl.BlockSpec((BLOCK_N, NUM_FEATURES), lambda j: (j, 0)),
        ],
        out_specs=[
            pl.BlockSpec((1, 1, BATCH), lambda j: (j, 0, 0)),
            pl.BlockSpec((1, 1, BATCH), lambda j: (j, 0, 0)),
        ],
        out_shape=[
            jax.ShapeDtypeStruct((NUM_BLOCKS, 1, BATCH), jnp.float32),
            jax.ShapeDtypeStruct((NUM_BLOCKS, 1, BATCH), jnp.float32),
        ],
        compiler_params=pltpu.CompilerParams(
            dimension_semantics=("parallel",)),
    )(inputs, t2d, fb16)
    loss = pl.pallas_call(
        _reduce_kernel,
        in_specs=[
            pl.BlockSpec((NUM_BLOCKS, 1, BATCH), lambda: (0, 0, 0)),
            pl.BlockSpec((NUM_BLOCKS, 1, BATCH), lambda: (0, 0, 0)),
        ],
        out_specs=pl.BlockSpec((1, 1), lambda: (0, 0)),
        out_shape=jax.ShapeDtypeStruct((1, 1), jnp.float32),
    )(s_part.reshape(NUM_BLOCKS, 1, BATCH), t_part.reshape(NUM_BLOCKS, 1, BATCH))
    return loss[0, 0]


# speedup vs baseline: 2.0790x; 1.1216x over previous
import jax
import jax.numpy as jnp
from jax.experimental import pallas as pl
from jax.experimental.pallas import tpu as pltpu
from jax.experimental.pallas import tpu_sc as plsc

NUM_SAMPLES = 100000
NUM_FEATURES = 64
BATCH = 1024
TEMP = 0.05
BLOCK_N = 2000
NUM_BLOCKS = NUM_SAMPLES // BLOCK_N
LOG2E_OVER_TEMP = 1.4426950408889634 / TEMP
LN2 = 0.6931471805599453
GATHER_WINDOW = 128


def _normalized(x):
    ss = jnp.sum(x * x, axis=1, keepdims=True)
    return x / jnp.maximum(jnp.sqrt(ss), 1e-12)


def _lse_kernel(x_ref, f_ref, o_ref):
    j = pl.program_id(0)
    xb = (_normalized(x_ref[...]) * LOG2E_OVER_TEMP).astype(jnp.bfloat16)
    fb = f_ref[...]
    l2 = jax.lax.dot_general(
        xb, fb, (((1,), (1,)), ((), ())),
        preferred_element_type=jnp.float32)
    eb = jnp.exp2(l2).astype(jnp.bfloat16)
    ones = jnp.ones((BLOCK_N, 128), jnp.bfloat16)
    part = jax.lax.dot_general(
        eb, ones, (((1,), (0,)), ((), ())),
        preferred_element_type=jnp.float32)

    @pl.when(j == 0)
    def _():
        o_ref[...] = part

    @pl.when(j > 0)
    def _():
        o_ref[...] += part


def _gather_rows(table, idx):
    @pl.kernel(
        out_type=jax.ShapeDtypeStruct((BATCH, 2 * NUM_FEATURES), table.dtype),
        mesh=plsc.VectorSubcoreMesh(
            core_axis_name="core", subcore_axis_name="subcore"))
    def _sc_kernel(t_hbm, i_hbm, o_hbm):
        def body(i_vmem, o_vmem):
            pltpu.sync_copy(t_hbm.at[i_vmem.at[0]], o_vmem)

        pltpu.emit_pipeline(
            body,
            grid=(BATCH // GATHER_WINDOW,),
            in_specs=[pl.BlockSpec((1, GATHER_WINDOW),
                                   index_map=lambda i: (0, i))],
            out_specs=[pl.BlockSpec((GATHER_WINDOW, 2 * NUM_FEATURES),
                                    index_map=lambda i: (i, 0))],
            core_axis_name="subcore",
            dimension_semantics=(pltpu.PARALLEL,),
        )(i_hbm, o_hbm)

    return _sc_kernel(table, idx.reshape(1, BATCH))


def _combine_kernel(x_ref, g_ref, p_ref, s_ref, out_ref):
    xn = _normalized(x_ref[...])
    g2 = g_ref[...]
    g = jnp.where(p_ref[...] == 1, g2[:, NUM_FEATURES:], g2[:, :NUM_FEATURES])
    t = jnp.sum(xn * g, axis=1) * (1.0 / TEMP)
    lse = jnp.log2(s_ref[...][:, 0]) * LN2
    out_ref[...] = jnp.mean(lse - t).reshape(1, 1)


@jax.jit
def kernel(inputs, targets, feature_memory, k, features):
    del feature_memory, k
    fb16 = features.astype(jnp.bfloat16)
    sumexp = pl.pallas_call(
        _lse_kernel,
        grid=(NUM_BLOCKS,),
        in_specs=[
            pl.BlockSpec((BATCH, NUM_FEATURES), lambda j: (0, 0)),
            pl.BlockSpec((BLOCK_N, NUM_FEATURES), lambda j: (j, 0)),
        ],
        out_specs=pl.BlockSpec((BATCH, 128), lambda j: (0, 0)),
        out_shape=jax.ShapeDtypeStruct((BATCH, 128), jnp.float32),
        compiler_params=pltpu.CompilerParams(
            dimension_semantics=("arbitrary",)),
    )(inputs, fb16)
    t32 = targets.astype(jnp.int32)
    gathered = _gather_rows(
        features.reshape(NUM_SAMPLES // 2, 2 * NUM_FEATURES), t32 // 2)
    parity = (t32 % 2).reshape(BATCH, 1)
    loss = pl.pallas_call(
        _combine_kernel,
        in_specs=[
            pl.BlockSpec((BATCH, NUM_FEATURES), lambda: (0, 0)),
            pl.BlockSpec((BATCH, 2 * NUM_FEATURES), lambda: (0, 0)),
            pl.BlockSpec((BATCH, 1), lambda: (0, 0)),
            pl.BlockSpec((BATCH, 128), lambda: (0, 0)),
        ],
        out_specs=pl.BlockSpec((1, 1), lambda: (0, 0)),
        out_shape=jax.ShapeDtypeStruct((1, 1), jnp.float32),
    )(inputs, gathered, parity, sumexp)
    return loss[0, 0]


# speedup vs baseline: 2.3168x; 1.1144x over previous
import jax
import jax.numpy as jnp
from jax.experimental import pallas as pl
from jax.experimental.pallas import tpu as pltpu
from jax.experimental.pallas import tpu_sc as plsc

NUM_SAMPLES = 100000
NUM_FEATURES = 64
BATCH = 1024
TEMP = 0.05
BLOCK_N = 2000
NUM_BLOCKS = NUM_SAMPLES // BLOCK_N
LOG2E_OVER_TEMP = 1.4426950408889634 / TEMP
LN2 = 0.6931471805599453
GATHER_WINDOW = 128


def _norm_kernel(x_ref, o_ref):
    x = x_ref[...]
    ss = jnp.sum(x * x, axis=1, keepdims=True)
    xn = x / jnp.maximum(jnp.sqrt(ss), 1e-12)
    o_ref[...] = (xn * LOG2E_OVER_TEMP).astype(jnp.bfloat16)


def _lse_kernel(xb_ref, f_ref, o_ref):
    j = pl.program_id(0)
    xb = xb_ref[...]
    fb = f_ref[...].astype(jnp.bfloat16)
    l2 = jax.lax.dot_general(
        xb, fb, (((1,), (1,)), ((), ())),
        preferred_element_type=jnp.float32)
    eb = jnp.exp2(l2.astype(jnp.bfloat16))
    ones = jnp.ones((BLOCK_N, 128), jnp.bfloat16)
    part = jax.lax.dot_general(
        eb, ones, (((1,), (0,)), ((), ())),
        preferred_element_type=jnp.float32)

    @pl.when(j == 0)
    def _():
        o_ref[...] = part

    @pl.when(j > 0)
    def _():
        o_ref[...] += part


def _gather_rows(table, idx):
    @pl.kernel(
        out_type=jax.ShapeDtypeStruct((BATCH, 2 * NUM_FEATURES), table.dtype),
        mesh=plsc.VectorSubcoreMesh(
            core_axis_name="core", subcore_axis_name="subcore"))
    def _sc_kernel(t_hbm, i_hbm, o_hbm):
        def body(i_vmem, o_vmem):
            pltpu.sync_copy(t_hbm.at[i_vmem.at[0]], o_vmem)

        pltpu.emit_pipeline(
            body,
            grid=(BATCH // GATHER_WINDOW,),
            in_specs=[pl.BlockSpec((1, GATHER_WINDOW),
                                   index_map=lambda i: (0, i))],
            out_specs=[pl.BlockSpec((GATHER_WINDOW, 2 * NUM_FEATURES),
                                    index_map=lambda i: (i, 0))],
            core_axis_name="subcore",
            dimension_semantics=(pltpu.PARALLEL,),
        )(i_hbm, o_hbm)

    return _sc_kernel(table, idx.reshape(1, BATCH))


def _combine_kernel(xb_ref, g_ref, p_ref, s_ref, out_ref):
    xb = xb_ref[...].astype(jnp.float32)
    g2 = g_ref[...]
    g = jnp.where(p_ref[...] == 1, g2[:, NUM_FEATURES:], g2[:, :NUM_FEATURES])
    t = jnp.sum(xb * g, axis=1) * LN2
    lse = jnp.log2(s_ref[...][:, 0]) * LN2
    out_ref[...] = jnp.mean(lse - t).reshape(1, 1)


@jax.jit
def kernel(inputs, targets, feature_memory, k, features):
    del feature_memory, k
    xb = pl.pallas_call(
        _norm_kernel,
        in_specs=[pl.BlockSpec((BATCH, NUM_FEATURES), lambda: (0, 0))],
        out_specs=pl.BlockSpec((BATCH, NUM_FEATURES), lambda: (0, 0)),
        out_shape=jax.ShapeDtypeStruct((BATCH, NUM_FEATURES), jnp.bfloat16),
    )(inputs)
    sumexp = pl.pallas_call(
        _lse_kernel,
        grid=(NUM_BLOCKS,),
        in_specs=[
            pl.BlockSpec((BATCH, NUM_FEATURES), lambda j: (0, 0)),
            pl.BlockSpec((BLOCK_N, NUM_FEATURES), lambda j: (j, 0)),
        ],
        out_specs=pl.BlockSpec((BATCH, 128), lambda j: (0, 0)),
        out_shape=jax.ShapeDtypeStruct((BATCH, 128), jnp.float32),
        compiler_params=pltpu.CompilerParams(
            dimension_semantics=("arbitrary",)),
    )(xb, features)
    t32 = targets.astype(jnp.int32)
    gathered = _gather_rows(
        features.reshape(NUM_SAMPLES // 2, 2 * NUM_FEATURES), t32 // 2)
    parity = (t32 % 2).reshape(BATCH, 1)
    loss = pl.pallas_call(
        _combine_kernel,
        in_specs=[
            pl.BlockSpec((BATCH, NUM_FEATURES), lambda: (0, 0)),
            pl.BlockSpec((BATCH, 2 * NUM_FEATURES), lambda: (0, 0)),
            pl.BlockSpec((BATCH, 1), lambda: (0, 0)),
            pl.BlockSpec((BATCH, 128), lambda: (0, 0)),
        ],
        out_specs=pl.BlockSpec((1, 1), lambda: (0, 0)),
        out_shape=jax.ShapeDtypeStruct((1, 1), jnp.float32),
    )(xb, gathered, parity, sumexp)
    return loss[0, 0]


# speedup vs baseline: 4.1530x; 1.7926x over previous
import jax
import jax.numpy as jnp
from jax.experimental import pallas as pl
from jax.experimental.pallas import tpu as pltpu
from jax.experimental.pallas import tpu_sc as plsc

NUM_SAMPLES = 100000
NUM_FEATURES = 64
BATCH = 1024
TEMP = 0.05
BLOCK_N = 4000
NUM_BLOCKS = NUM_SAMPLES // BLOCK_N
LOG2E_OVER_TEMP = 1.4426950408889634 / TEMP
LN2 = 0.6931471805599453
GATHER_WINDOW = 128


def _norm_kernel(x_ref, o_ref):
    x = x_ref[...]
    ss = jnp.sum(x * x, axis=1, keepdims=True)
    xn = x / jnp.maximum(jnp.sqrt(ss), 1e-12)
    o_ref[...] = (xn * LOG2E_OVER_TEMP).astype(jnp.bfloat16)


def _lse_kernel(xb_ref, f_ref, o_ref, fp_ref):
    j = pl.program_id(0)
    xb = xb_ref[...]
    f = f_ref[...]
    fb = f.astype(jnp.bfloat16)
    fp_ref[...] = jnp.concatenate(
        [f, jnp.zeros((BLOCK_N, 128 - NUM_FEATURES), jnp.float32)], axis=1)
    l2 = jax.lax.dot_general(
        xb, fb, (((1,), (1,)), ((), ())),
        preferred_element_type=jnp.float32)
    eb = jnp.exp2(l2.astype(jnp.bfloat16))
    nfull = BLOCK_N // 128
    part = eb[:, :128].astype(jnp.float32)
    for c in range(1, nfull):
        part += eb[:, c * 128:(c + 1) * 128]
    tail = BLOCK_N - nfull * 128
    if tail:
        part += jnp.concatenate(
            [eb[:, nfull * 128:],
             jnp.zeros((BATCH, 128 - tail), jnp.bfloat16)], axis=1)

    @pl.when(j == 0)
    def _():
        o_ref[...] = part

    @pl.when(j > 0)
    def _():
        o_ref[...] += part


def _gather_rows(table, idx):
    @pl.kernel(
        out_type=jax.ShapeDtypeStruct((BATCH, 128), table.dtype),
        mesh=plsc.VectorSubcoreMesh(
            core_axis_name="core", subcore_axis_name="subcore"))
    def _sc_kernel(t_hbm, i_hbm, o_hbm):
        def body(i_vmem, o_vmem):
            pltpu.sync_copy(t_hbm.at[i_vmem.at[0]], o_vmem)

        pltpu.emit_pipeline(
            body,
            grid=(BATCH // GATHER_WINDOW,),
            in_specs=[pl.BlockSpec((1, GATHER_WINDOW),
                                   index_map=lambda i: (0, i))],
            out_specs=[pl.BlockSpec((GATHER_WINDOW, 128),
                                    index_map=lambda i: (i, 0))],
            core_axis_name="subcore",
            dimension_semantics=(pltpu.PARALLEL,),
        )(i_hbm, o_hbm)

    return _sc_kernel(table, idx.reshape(1, BATCH))


def _combine_kernel(xb_ref, g_ref, s_ref, out_ref):
    xb = xb_ref[...].astype(jnp.float32)
    g = g_ref[...][:, :NUM_FEATURES]
    t = jnp.sum(xb * g, axis=1) * LN2
    lse = jnp.log2(jnp.sum(s_ref[...], axis=1)) * LN2
    out_ref[...] = jnp.mean(lse - t).reshape(1, 1)


@jax.jit
def kernel(inputs, targets, feature_memory, k, features):
    del feature_memory, k
    xb = pl.pallas_call(
        _norm_kernel,
        in_specs=[pl.BlockSpec((BATCH, NUM_FEATURES), lambda: (0, 0))],
        out_specs=pl.BlockSpec((BATCH, NUM_FEATURES), lambda: (0, 0)),
        out_shape=jax.ShapeDtypeStruct((BATCH, NUM_FEATURES), jnp.bfloat16),
    )(inputs)
    sumexp, fpack = pl.pallas_call(
        _lse_kernel,
        grid=(NUM_BLOCKS,),
        in_specs=[
            pl.BlockSpec((BATCH, NUM_FEATURES), lambda j: (0, 0)),
            pl.BlockSpec((BLOCK_N, NUM_FEATURES), lambda j: (j, 0)),
        ],
        out_specs=[
            pl.BlockSpec((BATCH, 128), lambda j: (0, 0)),
            pl.BlockSpec((BLOCK_N, 128), lambda j: (j, 0)),
        ],
        out_shape=[
            jax.ShapeDtypeStruct((BATCH, 128), jnp.float32),
            jax.ShapeDtypeStruct((NUM_SAMPLES, 128), jnp.float32),
        ],
        compiler_params=pltpu.CompilerParams(
            dimension_semantics=("arbitrary",)),
    )(xb, features)
    gathered = _gather_rows(fpack, targets.astype(jnp.int32))
    loss = pl.pallas_call(
        _combine_kernel,
        in_specs=[
            pl.BlockSpec((BATCH, NUM_FEATURES), lambda: (0, 0)),
            pl.BlockSpec((BATCH, 128), lambda: (0, 0)),
            pl.BlockSpec((BATCH, 128), lambda: (0, 0)),
        ],
        out_specs=pl.BlockSpec((1, 1), lambda: (0, 0)),
        out_shape=jax.ShapeDtypeStruct((1, 1), jnp.float32),
    )(xb, gathered, sumexp)
    return loss[0, 0]


# speedup vs baseline: 4.1630x; 1.0024x over previous
import jax
import jax.numpy as jnp
from jax.experimental import pallas as pl
from jax.experimental.pallas import tpu as pltpu
from jax.experimental.pallas import tpu_sc as plsc

NUM_SAMPLES = 100000
NUM_FEATURES = 64
BATCH = 1024
TEMP = 0.05
BLOCK_N = 4000
NUM_BLOCKS = NUM_SAMPLES // BLOCK_N
HALF_N = BLOCK_N // 2
LOG2E_OVER_TEMP = 1.4426950408889634 / TEMP
LN2 = 0.6931471805599453
GATHER_WINDOW = 128


def _norm_kernel(x_ref, o_ref):
    x = x_ref[...]
    ss = jnp.sum(x * x, axis=1, keepdims=True)
    xn = x / jnp.maximum(jnp.sqrt(ss), 1e-12)
    o_ref[...] = (xn * LOG2E_OVER_TEMP).astype(jnp.bfloat16)


def _lse_kernel(xb_ref, f_ref, o_ref, fp_ref):
    j = pl.program_id(0)
    xb = xb_ref[...]
    f = f_ref[...]
    fb = f.astype(jnp.bfloat16)
    fp_ref[...] = jnp.concatenate([f[:HALF_N], f[HALF_N:]], axis=1)
    l2 = jax.lax.dot_general(
        xb, fb, (((1,), (1,)), ((), ())),
        preferred_element_type=jnp.float32)
    eb = jnp.exp2(l2.astype(jnp.bfloat16))
    nfull = BLOCK_N // 128
    chunks = [eb[:, c * 128:(c + 1) * 128] for c in range(nfull)]
    tail = BLOCK_N - nfull * 128
    if tail:
        chunks.append(jnp.concatenate(
            [eb[:, nfull * 128:],
             jnp.zeros((BATCH, 128 - tail), jnp.bfloat16)], axis=1))
    while len(chunks) > 1:
        nxt = [a + b for a, b in zip(chunks[::2], chunks[1::2])]
        if len(chunks) % 2:
            nxt.append(chunks[-1])
        chunks = nxt
    part = chunks[0].astype(jnp.float32)

    @pl.when(j == 0)
    def _():
        o_ref[...] = part

    @pl.when(j > 0)
    def _():
        o_ref[...] += part


def _gather_rows(table, idx):
    @pl.kernel(
        out_type=jax.ShapeDtypeStruct((BATCH, 128), table.dtype),
        mesh=plsc.VectorSubcoreMesh(
            core_axis_name="core", subcore_axis_name="subcore"))
    def _sc_kernel(t_hbm, i_hbm, o_hbm):
        def body(i_vmem, o_vmem):
            pltpu.sync_copy(t_hbm.at[i_vmem.at[0]], o_vmem)

        pltpu.emit_pipeline(
            body,
            grid=(BATCH // GATHER_WINDOW,),
            in_specs=[pl.BlockSpec((1, GATHER_WINDOW),
                                   index_map=lambda i: (0, i))],
            out_specs=[pl.BlockSpec((GATHER_WINDOW, 128),
                                    index_map=lambda i: (i, 0))],
            core_axis_name="subcore",
            dimension_semantics=(pltpu.PARALLEL,),
        )(i_hbm, o_hbm)

    return _sc_kernel(table, idx.reshape(1, BATCH))


def _combine_kernel(xb_ref, g_ref, m_ref, s_ref, out_ref):
    xb = xb_ref[...].astype(jnp.float32)
    gp = g_ref[...]
    m = m_ref[...]
    g = gp[:, :NUM_FEATURES] * m + gp[:, NUM_FEATURES:] * (1.0 - m)
    t = jnp.sum(xb * g, axis=1) * LN2
    lse = jnp.log2(jnp.sum(s_ref[...], axis=1)) * LN2
    out_ref[...] = jnp.mean(lse - t).reshape(1, 1)


@jax.jit
def kernel(inputs, targets, feature_memory, k, features):
    del feature_memory, k
    xb = pl.pallas_call(
        _norm_kernel,
        in_specs=[pl.BlockSpec((BATCH, NUM_FEATURES), lambda: (0, 0))],
        out_specs=pl.BlockSpec((BATCH, NUM_FEATURES), lambda: (0, 0)),
        out_shape=jax.ShapeDtypeStruct((BATCH, NUM_FEATURES), jnp.bfloat16),
    )(inputs)
    sumexp, fpack = pl.pallas_call(
        _lse_kernel,
        grid=(NUM_BLOCKS,),
        in_specs=[
            pl.BlockSpec((BATCH, NUM_FEATURES), lambda j: (0, 0)),
            pl.BlockSpec((BLOCK_N, NUM_FEATURES), lambda j: (j, 0)),
        ],
        out_specs=[
            pl.BlockSpec((BATCH, 128), lambda j: (0, 0)),
            pl.BlockSpec((HALF_N, 128), lambda j: (j, 0)),
        ],
        out_shape=[
            jax.ShapeDtypeStruct((BATCH, 128), jnp.float32),
            jax.ShapeDtypeStruct((NUM_SAMPLES // 2, 128), jnp.float32),
        ],
        compiler_params=pltpu.CompilerParams(
            dimension_semantics=("arbitrary",)),
    )(xb, features)
    tg = targets.astype(jnp.int32)
    pidx = (tg // BLOCK_N) * HALF_N + tg % HALF_N
    low_half = ((tg % BLOCK_N) < HALF_N).astype(jnp.float32).reshape(BATCH, 1)
    gathered = _gather_rows(fpack, pidx)
    loss = pl.pallas_call(
        _combine_kernel,
        in_specs=[
            pl.BlockSpec((BATCH, NUM_FEATURES), lambda: (0, 0)),
            pl.BlockSpec((BATCH, 128), lambda: (0, 0)),
            pl.BlockSpec((BATCH, 1), lambda: (0, 0)),
            pl.BlockSpec((BATCH, 128), lambda: (0, 0)),
        ],
        out_specs=pl.BlockSpec((1, 1), lambda: (0, 0)),
        out_shape=jax.ShapeDtypeStruct((1, 1), jnp.float32),
    )(xb, gathered, low_half, sumexp)
    return loss[0, 0]
